# Initial kernel scaffold; baseline (speedup 1.0000x reference)
import jax
import jax.numpy as jnp
from jax.experimental import pallas as pl


def kernel(x, edge_index, W0l, W0r, b0, bn_gamma, bn_beta, bn_mean, bn_var, W1l, W1r, b1):
    raise NotImplementedError("write your pallas kernel here")


# speedup vs baseline: 4.5244x; 4.5244x over previous
import functools

import jax
import jax.numpy as jnp
from jax import lax
from jax.experimental import pallas as pl
from jax.experimental.pallas import tpu as pltpu
from jax.experimental.pallas import tpu_sc as plsc

N = 10000
E = 320000
D = 128
DA = 144
NPAD = 10240
NC = 2
NS = 16
NW = NC * NS
EPW = E // NW
CH = 80
NCH = EPW // CH
RPS = NPAD // NS


def _sc_agg_body(xa, src, dst, zeros, out, srcv, dstv, rows, acc):
    cid = lax.axis_index("c")
    sid = lax.axis_index("s")
    wid = sid * NC + cid

    pltpu.sync_copy(zeros, rows)
    base_r = sid * RPS

    def zfill(i, carry):
        r = pl.multiple_of(base_r + i * CH, 8)
        pltpu.sync_copy(rows, acc.at[pl.ds(r, CH)])
        return carry

    lax.fori_loop(0, RPS // CH, zfill, 0)
    plsc.subcore_barrier()

    ebase = wid * EPW

    def step(k, carry):
        off = pl.multiple_of(ebase + k * CH, 8)
        pltpu.sync_copy(src.at[pl.ds(off, CH)], srcv)
        pltpu.sync_copy(dst.at[pl.ds(off, CH)], dstv)
        pltpu.sync_copy(xa.at[srcv], rows)
        pltpu.sync_copy(rows, acc.at[dstv], add=True)
        return carry

    lax.fori_loop(0, NCH, step, 0)
    plsc.subcore_barrier()

    def wb(i, carry):
        r = pl.multiple_of(base_r + i * CH, 8)
        pltpu.sync_copy(acc.at[pl.ds(r, CH)], rows)
        pltpu.sync_copy(rows, out.at[cid, pl.ds(r, CH)])
        return carry

    lax.fori_loop(0, RPS // CH, wb, 0)


_sc_agg = functools.partial(
    pl.kernel,
    mesh=plsc.VectorSubcoreMesh(core_axis_name="c", subcore_axis_name="s"),
    out_type=jax.ShapeDtypeStruct((NC, NPAD, DA), jnp.float32),
    scratch_types=[
        pltpu.VMEM((CH,), jnp.int32),
        pltpu.VMEM((CH,), jnp.int32),
        pltpu.VMEM((CH, DA), jnp.float32),
        pltpu.VMEM_SHARED((NPAD, DA), jnp.float32),
    ],
    compiler_params=pltpu.CompilerParams(use_tc_tiling_on_sc=False),
)(_sc_agg_body)


def _tc_body(agg_ref, xa_ref, a_ref, b_ref, bias_ref, out_ref, *, relu, aug):
    a = agg_ref[0] + agg_ref[1]
    deg = jnp.maximum(a[:, D:D + 1], 1.0)
    m = a[:, :D] / deg
    y = jnp.dot(m, a_ref[:], preferred_element_type=jnp.float32)
    y = y + jnp.dot(xa_ref[:, :D], b_ref[:], preferred_element_type=jnp.float32)
    y = y + bias_ref[:]
    if relu:
        y = jnp.maximum(y, 0.0)
    if aug:
        out_ref[:, :D] = y
        out_ref[:, D:] = jnp.ones((y.shape[0], DA - D), jnp.float32)
    else:
        out_ref[:] = y


def _dense_layer(s_pair, xa, a_w, b_w, bias, relu, aug):
    br = 256
    dout = DA if aug else D
    return pl.pallas_call(
        functools.partial(_tc_body, relu=relu, aug=aug),
        grid=(NPAD // br,),
        in_specs=[
            pl.BlockSpec((NC, br, DA), lambda i: (0, i, 0)),
            pl.BlockSpec((br, DA), lambda i: (i, 0)),
            pl.BlockSpec((D, D), lambda i: (0, 0)),
            pl.BlockSpec((D, D), lambda i: (0, 0)),
            pl.BlockSpec((1, D), lambda i: (0, 0)),
        ],
        out_specs=pl.BlockSpec((br, dout), lambda i: (i, 0)),
        out_shape=jax.ShapeDtypeStruct((NPAD, dout), jnp.float32),
    )(s_pair, xa, a_w, b_w, bias)


def kernel(x, edge_index, W0l, W0r, b0, bn_gamma, bn_beta, bn_mean, bn_var, W1l, W1r, b1):
    src = edge_index[0]
    dst = edge_index[1]

    g = bn_gamma / jnp.sqrt(bn_var + 1e-5)
    c = bn_beta - bn_mean * g
    a0 = W0l.T * g
    b0w = W0r.T * g
    bias0 = (b0 * g + c)[None, :]
    a1 = W1l.T
    b1w = W1r.T
    bias1 = b1[None, :]

    xa = jnp.zeros((NPAD, DA), jnp.float32)
    xa = xa.at[:N, :D].set(x).at[:N, D:].set(1.0)
    zeros = jnp.zeros((CH, DA), jnp.float32)

    agg0 = _sc_agg(xa, src, dst, zeros)
    ha = _dense_layer(agg0, xa, a0, b0w, bias0, relu=True, aug=True)
    agg1 = _sc_agg(ha, src, dst, zeros)
    logit = _dense_layer(agg1, ha, a1, b1w, bias1, relu=False, aug=False)

    return (logit[:N], ha[:N, :D])


# speedup vs baseline: 5.5289x; 1.2220x over previous
import functools

import jax
import jax.numpy as jnp
from jax import lax
from jax.experimental import pallas as pl
from jax.experimental.pallas import tpu as pltpu
from jax.experimental.pallas import tpu_sc as plsc

N = 10000
E = 320000
D = 128
DA = 144
NPAD = 10240
NC = 2
NS = 16
NW = NC * NS
EPAD = 322560
EPW = EPAD // NW
CH = 80
NCH = EPW // CH
NB = 3
NGRP = NCH // NB
RPS = NPAD // NS


def _sc_agg_body(xa, src, dst, zeros, out, sgi, dgi, rows0, rows1, rows2,
                 acc, sem0, sem1, sem2, semi):
    rows = (rows0, rows1, rows2)
    sems = (sem0, sem1, sem2)
    cid = lax.axis_index("c")
    sid = lax.axis_index("s")
    wid = sid * NC + cid

    pltpu.sync_copy(zeros, rows[0])
    base_r = sid * RPS

    def zfill(i, carry):
        r = pl.multiple_of(base_r + i * CH, 8)
        pltpu.sync_copy(rows[0], acc.at[pl.ds(r, CH)])
        return carry

    lax.fori_loop(0, RPS // CH, zfill, 0)
    plsc.subcore_barrier()

    pltpu.sync_copy(src.at[wid, pl.ds(0, NB)], sgi.at[0])
    pltpu.sync_copy(dst.at[wid, pl.ds(0, NB)], dgi.at[0])
    pltpu.make_async_copy(src.at[wid, pl.ds(NB, NB)], sgi.at[1], semi).start()
    pltpu.make_async_copy(dst.at[wid, pl.ds(NB, NB)], dgi.at[1], semi).start()
    for b in range(NB):
        pltpu.make_async_copy(xa.at[sgi.at[0, b]], rows[b], sems[b]).start()

    def pair(p, carry):
        for sl in range(2):
            g = 2 * p + sl
            nsl = 1 - sl

            @pl.when(g + 1 < NGRP)
            def _():
                pltpu.make_async_copy(
                    src.at[wid, pl.ds((g + 1) * NB, NB)], sgi.at[nsl],
                    semi).wait()
                pltpu.make_async_copy(
                    dst.at[wid, pl.ds((g + 1) * NB, NB)], dgi.at[nsl],
                    semi).wait()

            for b in range(NB):
                pltpu.make_async_copy(
                    xa.at[sgi.at[sl, b]], rows[b], sems[b]).wait()
                pltpu.sync_copy(rows[b], acc.at[dgi.at[sl, b]], add=True)

                @pl.when(g + 1 < NGRP)
                def _():
                    pltpu.make_async_copy(
                        xa.at[sgi.at[nsl, b]], rows[b], sems[b]).start()

            @pl.when(g + 2 < NGRP)
            def _():
                pltpu.make_async_copy(
                    src.at[wid, pl.ds((g + 2) * NB, NB)], sgi.at[sl],
                    semi).start()
                pltpu.make_async_copy(
                    dst.at[wid, pl.ds((g + 2) * NB, NB)], dgi.at[sl],
                    semi).start()

        return carry

    lax.fori_loop(0, NGRP // 2, pair, 0)
    plsc.subcore_barrier()

    def wb(i, carry):
        r = pl.multiple_of(base_r + i * CH, 8)
        pltpu.sync_copy(acc.at[pl.ds(r, CH)], rows[0])
        pltpu.sync_copy(rows[0], out.at[cid, pl.ds(r, CH)])
        return carry

    lax.fori_loop(0, RPS // CH, wb, 0)


_sc_agg = functools.partial(
    pl.kernel,
    mesh=plsc.VectorSubcoreMesh(core_axis_name="c", subcore_axis_name="s"),
    out_type=jax.ShapeDtypeStruct((NC, NPAD, DA), jnp.float32),
    scratch_types=[
        pltpu.VMEM((2, NB, CH), jnp.int32),
        pltpu.VMEM((2, NB, CH), jnp.int32),
    ] + [pltpu.VMEM((CH, DA), jnp.float32) for _ in range(NB)] + [
        pltpu.VMEM_SHARED((NPAD, DA), jnp.float32),
    ] + [pltpu.SemaphoreType.DMA for _ in range(NB + 1)],
    compiler_params=pltpu.CompilerParams(use_tc_tiling_on_sc=False),
)(_sc_agg_body)


def _tc_body(agg_ref, xa_ref, a_ref, b_ref, bias_ref, out_ref, *, relu, aug):
    a = agg_ref[0] + agg_ref[1]
    deg = jnp.maximum(a[:, D:D + 1], 1.0)
    m = a[:, :D] / deg
    y = jnp.dot(m, a_ref[:], preferred_element_type=jnp.float32)
    y = y + jnp.dot(xa_ref[:, :D], b_ref[:], preferred_element_type=jnp.float32)
    y = y + bias_ref[:]
    if relu:
        y = jnp.maximum(y, 0.0)
    if aug:
        out_ref[:, :D] = y
        out_ref[:, D:] = jnp.ones((y.shape[0], DA - D), jnp.float32)
    else:
        out_ref[:] = y


def _dense_layer(s_pair, xa, a_w, b_w, bias, relu, aug):
    br = 256
    dout = DA if aug else D
    return pl.pallas_call(
        functools.partial(_tc_body, relu=relu, aug=aug),
        grid=(NPAD // br,),
        in_specs=[
            pl.BlockSpec((NC, br, DA), lambda i: (0, i, 0)),
            pl.BlockSpec((br, DA), lambda i: (i, 0)),
            pl.BlockSpec((D, D), lambda i: (0, 0)),
            pl.BlockSpec((D, D), lambda i: (0, 0)),
            pl.BlockSpec((1, D), lambda i: (0, 0)),
        ],
        out_specs=pl.BlockSpec((br, dout), lambda i: (i, 0)),
        out_shape=jax.ShapeDtypeStruct((NPAD, dout), jnp.float32),
    )(s_pair, xa, a_w, b_w, bias)


def kernel(x, edge_index, W0l, W0r, b0, bn_gamma, bn_beta, bn_mean, bn_var, W1l, W1r, b1):
    src = edge_index[0]
    dst = edge_index[1]

    g = bn_gamma / jnp.sqrt(bn_var + 1e-5)
    c = bn_beta - bn_mean * g
    a0 = W0l.T * g
    b0w = W0r.T * g
    bias0 = (b0 * g + c)[None, :]
    a1 = W1l.T
    b1w = W1r.T
    bias1 = b1[None, :]

    xa = jnp.zeros((NPAD, DA), jnp.float32)
    xa = xa.at[:N, :D].set(x).at[:N, D:].set(1.0)
    zeros = jnp.zeros((CH, DA), jnp.float32)

    pad_e = EPAD - E
    src_c = jnp.concatenate(
        [src, jnp.zeros((pad_e,), jnp.int32)]).reshape(NW, NCH, CH)
    dst_c = jnp.concatenate(
        [dst, jnp.full((pad_e,), NPAD - 1, jnp.int32)]).reshape(NW, NCH, CH)

    agg0 = _sc_agg(xa, src_c, dst_c, zeros)
    ha = _dense_layer(agg0, xa, a0, b0w, bias0, relu=True, aug=True)
    agg1 = _sc_agg(ha, src_c, dst_c, zeros)
    logit = _dense_layer(agg1, ha, a1, b1w, bias1, relu=False, aug=False)

    return (logit[:N], ha[:N, :D])


# speedup vs baseline: 7.3876x; 1.3362x over previous
import functools

import jax
import jax.numpy as jnp
from jax import lax
from jax.experimental import pallas as pl
from jax.experimental.pallas import tpu as pltpu
from jax.experimental.pallas import tpu_sc as plsc

N = 10000
E = 320000
D = 128
DG = 16
NPAD = 10240
NC = 2
NS = 16
EPAD = 322560
CH = 80
TOTCH = EPAD // CH
NB = 3
FAST_CID = 0
CPW_FAST = 168
CPW_SLOW = 84
RPS = NPAD // NS


def _sc_agg_body(x, src, dst, zeros, ones16, zeros16, outs, sgi, dgi,
                 rows0, rows1, rows2, acc, acc_d, ones_v, dbuf,
                 sem0, sem1, sem2, semi, *, with_deg):
    rows = (rows0, rows1, rows2)
    sems = (sem0, sem1, sem2)
    if with_deg:
        out, out_d = outs
    else:
        out = outs
    cid = lax.axis_index("c")
    sid = lax.axis_index("s")
    fast = cid == FAST_CID
    cb = jnp.where(fast, sid * CPW_FAST, NS * CPW_FAST + sid * CPW_SLOW)
    ngrp = jnp.where(fast, CPW_FAST // NB, CPW_SLOW // NB)
    npair = jnp.where(fast, CPW_FAST // (2 * NB), CPW_SLOW // (2 * NB))

    pltpu.sync_copy(zeros, rows[0])
    if with_deg:
        pltpu.sync_copy(ones16, ones_v)
        pltpu.sync_copy(zeros16, dbuf)
    base_r = sid * RPS

    def zfill(i, carry):
        r = pl.multiple_of(base_r + i * CH, 8)
        pltpu.sync_copy(rows[0], acc.at[pl.ds(r, CH)])
        if with_deg:
            pltpu.sync_copy(dbuf, acc_d.at[pl.ds(r, CH)])
        return carry

    lax.fori_loop(0, RPS // CH, zfill, 0)
    plsc.subcore_barrier()

    pltpu.sync_copy(src.at[pl.ds(cb, NB)], sgi.at[0])
    pltpu.sync_copy(dst.at[pl.ds(cb, NB)], dgi.at[0])
    pltpu.make_async_copy(src.at[pl.ds(cb + NB, NB)], sgi.at[1], semi).start()
    pltpu.make_async_copy(dst.at[pl.ds(cb + NB, NB)], dgi.at[1], semi).start()
    for b in range(NB):
        pltpu.make_async_copy(x.at[sgi.at[0, b]], rows[b], sems[b]).start()

    def pair(p, carry):
        for sl in range(2):
            g = 2 * p + sl
            nsl = 1 - sl

            @pl.when(g + 1 < ngrp)
            def _():
                pltpu.make_async_copy(
                    src.at[pl.ds(cb + (g + 1) * NB, NB)], sgi.at[nsl],
                    semi).wait()
                pltpu.make_async_copy(
                    dst.at[pl.ds(cb + (g + 1) * NB, NB)], dgi.at[nsl],
                    semi).wait()

            for b in range(NB):
                pltpu.make_async_copy(
                    x.at[sgi.at[sl, b]], rows[b], sems[b]).wait()
                pltpu.sync_copy(rows[b], acc.at[dgi.at[sl, b]], add=True)
                if with_deg:
                    pltpu.sync_copy(ones_v, acc_d.at[dgi.at[sl, b]],
                                    add=True)

                @pl.when(g + 1 < ngrp)
                def _():
                    pltpu.make_async_copy(
                        x.at[sgi.at[nsl, b]], rows[b], sems[b]).start()

            @pl.when(g + 2 < ngrp)
            def _():
                pltpu.make_async_copy(
                    src.at[pl.ds(cb + (g + 2) * NB, NB)], sgi.at[sl],
                    semi).start()
                pltpu.make_async_copy(
                    dst.at[pl.ds(cb + (g + 2) * NB, NB)], dgi.at[sl],
                    semi).start()

        return carry

    lax.fori_loop(0, npair, pair, 0)
    plsc.subcore_barrier()

    def wb(i, carry):
        r = pl.multiple_of(base_r + i * CH, 8)
        pltpu.sync_copy(acc.at[pl.ds(r, CH)], rows[0])
        pltpu.sync_copy(rows[0], out.at[cid, pl.ds(r, CH)])
        if with_deg:
            pltpu.sync_copy(acc_d.at[pl.ds(r, CH)], dbuf)
            pltpu.sync_copy(dbuf, out_d.at[cid, pl.ds(r, CH)])
        return carry

    lax.fori_loop(0, RPS // CH, wb, 0)


def _sc_agg_body_deg(x, src, dst, zeros, ones16, zeros16, out, out_d, sgi,
                     dgi, rows0, rows1, rows2, acc, acc_d, ones_v, dbuf,
                     sem0, sem1, sem2, semi):
    _sc_agg_body(x, src, dst, zeros, ones16, zeros16, (out, out_d), sgi,
                 dgi, rows0, rows1, rows2, acc, acc_d, ones_v, dbuf,
                 sem0, sem1, sem2, semi, with_deg=True)


def _sc_agg_body_nodeg(x, src, dst, zeros, out, sgi, dgi, rows0, rows1,
                       rows2, acc, sem0, sem1, sem2, semi):
    _sc_agg_body(x, src, dst, zeros, None, None, out, sgi, dgi, rows0,
                 rows1, rows2, acc, None, None, None, sem0, sem1, sem2,
                 semi, with_deg=False)


_sc_agg_deg = functools.partial(
    pl.kernel,
    mesh=plsc.VectorSubcoreMesh(core_axis_name="c", subcore_axis_name="s"),
    out_type=(jax.ShapeDtypeStruct((NC, NPAD, D), jnp.float32),
              jax.ShapeDtypeStruct((NC, NPAD, DG), jnp.float32)),
    scratch_types=[
        pltpu.VMEM((2, NB, CH), jnp.int32),
        pltpu.VMEM((2, NB, CH), jnp.int32),
    ] + [pltpu.VMEM((CH, D), jnp.float32) for _ in range(NB)] + [
        pltpu.VMEM_SHARED((NPAD, D), jnp.float32),
        pltpu.VMEM_SHARED((NPAD, DG), jnp.float32),
        pltpu.VMEM((CH, DG), jnp.float32),
        pltpu.VMEM((CH, DG), jnp.float32),
    ] + [pltpu.SemaphoreType.DMA for _ in range(NB + 1)],
    compiler_params=pltpu.CompilerParams(use_tc_tiling_on_sc=False),
)(_sc_agg_body_deg)

_sc_agg_nodeg = functools.partial(
    pl.kernel,
    mesh=plsc.VectorSubcoreMesh(core_axis_name="c", subcore_axis_name="s"),
    out_type=jax.ShapeDtypeStruct((NC, NPAD, D), jnp.float32),
    scratch_types=[
        pltpu.VMEM((2, NB, CH), jnp.int32),
        pltpu.VMEM((2, NB, CH), jnp.int32),
    ] + [pltpu.VMEM((CH, D), jnp.float32) for _ in range(NB)] + [
        pltpu.VMEM_SHARED((NPAD, D), jnp.float32),
    ] + [pltpu.SemaphoreType.DMA for _ in range(NB + 1)],
    compiler_params=pltpu.CompilerParams(use_tc_tiling_on_sc=False),
)(_sc_agg_body_nodeg)


def _tc_body(sums_ref, degs_ref, x_ref, a_ref, b_ref, bias_ref, out_ref, *,
             relu):
    s = sums_ref[0] + sums_ref[1]
    deg = jnp.maximum(degs_ref[0, :, :1] + degs_ref[1, :, :1], 1.0)
    m = s / deg
    y = jnp.dot(m, a_ref[:], preferred_element_type=jnp.float32)
    y = y + jnp.dot(x_ref[:], b_ref[:], preferred_element_type=jnp.float32)
    y = y + bias_ref[:]
    if relu:
        y = jnp.maximum(y, 0.0)
    out_ref[:] = y


def _dense_layer(sums, degs, x, a_w, b_w, bias, relu):
    br = 200
    return pl.pallas_call(
        functools.partial(_tc_body, relu=relu),
        grid=(N // br,),
        in_specs=[
            pl.BlockSpec((NC, br, D), lambda i: (0, i, 0)),
            pl.BlockSpec((NC, br, DG), lambda i: (0, i, 0)),
            pl.BlockSpec((br, D), lambda i: (i, 0)),
            pl.BlockSpec((D, D), lambda i: (0, 0)),
            pl.BlockSpec((D, D), lambda i: (0, 0)),
            pl.BlockSpec((1, D), lambda i: (0, 0)),
        ],
        out_specs=pl.BlockSpec((br, D), lambda i: (i, 0)),
        out_shape=jax.ShapeDtypeStruct((N, D), jnp.float32),
    )(sums, degs, x, a_w, b_w, bias)


def kernel(x, edge_index, W0l, W0r, b0, bn_gamma, bn_beta, bn_mean, bn_var, W1l, W1r, b1):
    src = edge_index[0]
    dst = edge_index[1]

    g = bn_gamma / jnp.sqrt(bn_var + 1e-5)
    c = bn_beta - bn_mean * g
    a0 = W0l.T * g
    b0w = W0r.T * g
    bias0 = (b0 * g + c)[None, :]
    a1 = W1l.T
    b1w = W1r.T
    bias1 = b1[None, :]

    zeros = jnp.zeros((CH, D), jnp.float32)
    ones16 = jnp.ones((CH, DG), jnp.float32)
    zeros16 = jnp.zeros((CH, DG), jnp.float32)

    pad_e = EPAD - E
    src_c = jnp.concatenate(
        [src, jnp.zeros((pad_e,), jnp.int32)]).reshape(TOTCH, CH)
    dst_c = jnp.concatenate(
        [dst, jnp.full((pad_e,), NPAD - 1, jnp.int32)]).reshape(TOTCH, CH)

    sums0, degs = _sc_agg_deg(x, src_c, dst_c, zeros, ones16, zeros16)
    h = _dense_layer(sums0, degs, x, a0, b0w, bias0, relu=True)
    sums1 = _sc_agg_nodeg(h, src_c, dst_c, zeros)
    logit = _dense_layer(sums1, degs, h, a1, b1w, bias1, relu=False)

    return (logit, h)


# speedup vs baseline: 7.4094x; 1.0030x over previous
import functools

import jax
import jax.numpy as jnp
from jax import lax
from jax.experimental import pallas as pl
from jax.experimental.pallas import tpu as pltpu
from jax.experimental.pallas import tpu_sc as plsc

N = 10000
E = 320000
D = 128
DG = 16
NPAD = 10240
NC = 2
NS = 16
EPAD = 322560
CH = 80
TOTCH = EPAD // CH
NB = 3
FAST_CID = 0
CPW_FAST = 168
CPW_SLOW = 84
RPS = NPAD // NS


def _sc_agg_body(x, src, dst, zeros, ones16, zeros16, outs, sgi, dgi,
                 rows0, rows1, rows2, zbuf, acc, acc_d, ones_v, dbuf,
                 sem0, sem1, sem2, semi, semz, semd, *, with_deg):
    rows = (rows0, rows1, rows2)
    sems = (sem0, sem1, sem2)
    if with_deg:
        out, out_d = outs
    else:
        out = outs
    cid = lax.axis_index("c")
    sid = lax.axis_index("s")
    fast = cid == FAST_CID
    cb = jnp.where(fast, sid * CPW_FAST, NS * CPW_FAST + sid * CPW_SLOW)
    ngrp = jnp.where(fast, CPW_FAST // NB, CPW_SLOW // NB)
    npair = jnp.where(fast, CPW_FAST // (2 * NB), CPW_SLOW // (2 * NB))
    base_r = sid * RPS

    pltpu.sync_copy(src.at[pl.ds(cb, NB)], sgi.at[0])
    pltpu.sync_copy(dst.at[pl.ds(cb, NB)], dgi.at[0])
    pltpu.make_async_copy(src.at[pl.ds(cb + NB, NB)], sgi.at[1], semi).start()
    pltpu.make_async_copy(dst.at[pl.ds(cb + NB, NB)], dgi.at[1], semi).start()
    for b in range(NB):
        pltpu.make_async_copy(x.at[sgi.at[0, b]], rows[b], sems[b]).start()

    pltpu.sync_copy(zeros.at[pl.ds(0, 16)], zbuf)
    if with_deg:
        pltpu.sync_copy(ones16, ones_v)
        pltpu.sync_copy(zeros16, dbuf)
        for i in range(RPS // CH):
            r = base_r + i * CH
            pltpu.make_async_copy(dbuf, acc_d.at[pl.ds(r, CH)], semd).start()
    nz = RPS // 16
    for w in range(0, nz, 8):
        for i in range(w, w + 8):
            r = base_r + i * 16
            pltpu.make_async_copy(zbuf, acc.at[pl.ds(r, 16)], semz).start()
        for i in range(w, w + 8):
            r = base_r + i * 16
            pltpu.make_async_copy(zbuf, acc.at[pl.ds(r, 16)], semz).wait()
    if with_deg:
        for i in range(RPS // CH):
            r = base_r + i * CH
            pltpu.make_async_copy(dbuf, acc_d.at[pl.ds(r, CH)], semd).wait()
    plsc.subcore_barrier()

    def pair(p, carry):
        for sl in range(2):
            g = 2 * p + sl
            nsl = 1 - sl

            @pl.when(g + 1 < ngrp)
            def _():
                pltpu.make_async_copy(
                    src.at[pl.ds(cb + (g + 1) * NB, NB)], sgi.at[nsl],
                    semi).wait()
                pltpu.make_async_copy(
                    dst.at[pl.ds(cb + (g + 1) * NB, NB)], dgi.at[nsl],
                    semi).wait()

            for b in range(NB):
                pltpu.make_async_copy(
                    x.at[sgi.at[sl, b]], rows[b], sems[b]).wait()
                pltpu.sync_copy(rows[b], acc.at[dgi.at[sl, b]], add=True)
                if with_deg:
                    pltpu.sync_copy(ones_v, acc_d.at[dgi.at[sl, b]],
                                    add=True)

                @pl.when(g + 1 < ngrp)
                def _():
                    pltpu.make_async_copy(
                        x.at[sgi.at[nsl, b]], rows[b], sems[b]).start()

            @pl.when(g + 2 < ngrp)
            def _():
                pltpu.make_async_copy(
                    src.at[pl.ds(cb + (g + 2) * NB, NB)], sgi.at[sl],
                    semi).start()
                pltpu.make_async_copy(
                    dst.at[pl.ds(cb + (g + 2) * NB, NB)], dgi.at[sl],
                    semi).start()

        return carry

    lax.fori_loop(0, npair, pair, 0)
    plsc.subcore_barrier()

    nwb = RPS // CH
    for i in range(nwb):
        b = i % NB
        if i >= NB:
            pltpu.make_async_copy(
                rows[b], out.at[cid, pl.ds(base_r + (i - NB) * CH, CH)],
                sems[b]).wait()
        pltpu.sync_copy(acc.at[pl.ds(base_r + i * CH, CH)], rows[b])
        pltpu.make_async_copy(
            rows[b], out.at[cid, pl.ds(base_r + i * CH, CH)],
            sems[b]).start()
    if with_deg:
        dbs = (dbuf, ones_v)
        dsems = (semd, semz)
        for i in range(nwb):
            b = i % 2
            if i >= 2:
                pltpu.make_async_copy(
                    dbs[b], out_d.at[cid, pl.ds(base_r + (i - 2) * CH, CH)],
                    dsems[b]).wait()
            pltpu.sync_copy(acc_d.at[pl.ds(base_r + i * CH, CH)], dbs[b])
            pltpu.make_async_copy(
                dbs[b], out_d.at[cid, pl.ds(base_r + i * CH, CH)],
                dsems[b]).start()
        for i in range(nwb - 2, nwb):
            b = i % 2
            pltpu.make_async_copy(
                dbs[b], out_d.at[cid, pl.ds(base_r + i * CH, CH)],
                dsems[b]).wait()
    for i in range(nwb - NB, nwb):
        b = i % NB
        pltpu.make_async_copy(
            rows[b], out.at[cid, pl.ds(base_r + i * CH, CH)],
            sems[b]).wait()


def _sc_agg_body_deg(x, src, dst, zeros, ones16, zeros16, out, out_d, sgi,
                     dgi, rows0, rows1, rows2, zbuf, acc, acc_d, ones_v,
                     dbuf, sem0, sem1, sem2, semi, semz, semd):
    _sc_agg_body(x, src, dst, zeros, ones16, zeros16, (out, out_d), sgi,
                 dgi, rows0, rows1, rows2, zbuf, acc, acc_d, ones_v, dbuf,
                 sem0, sem1, sem2, semi, semz, semd, with_deg=True)


def _sc_agg_body_nodeg(x, src, dst, zeros, out, sgi, dgi, rows0, rows1,
                       rows2, zbuf, acc, sem0, sem1, sem2, semi, semz):
    _sc_agg_body(x, src, dst, zeros, None, None, out, sgi, dgi, rows0,
                 rows1, rows2, zbuf, acc, None, None, None, sem0, sem1,
                 sem2, semi, semz, None, with_deg=False)


_sc_agg_deg = functools.partial(
    pl.kernel,
    mesh=plsc.VectorSubcoreMesh(core_axis_name="c", subcore_axis_name="s"),
    out_type=(jax.ShapeDtypeStruct((NC, NPAD, D), jnp.float32),
              jax.ShapeDtypeStruct((NC, NPAD, DG), jnp.float32)),
    scratch_types=[
        pltpu.VMEM((2, NB, CH), jnp.int32),
        pltpu.VMEM((2, NB, CH), jnp.int32),
    ] + [pltpu.VMEM((CH, D), jnp.float32) for _ in range(NB)] + [
        pltpu.VMEM((16, D), jnp.float32),
        pltpu.VMEM_SHARED((NPAD, D), jnp.float32),
        pltpu.VMEM_SHARED((NPAD, DG), jnp.float32),
        pltpu.VMEM((CH, DG), jnp.float32),
        pltpu.VMEM((CH, DG), jnp.float32),
    ] + [pltpu.SemaphoreType.DMA for _ in range(NB + 3)],
    compiler_params=pltpu.CompilerParams(use_tc_tiling_on_sc=False),
)(_sc_agg_body_deg)

_sc_agg_nodeg = functools.partial(
    pl.kernel,
    mesh=plsc.VectorSubcoreMesh(core_axis_name="c", subcore_axis_name="s"),
    out_type=jax.ShapeDtypeStruct((NC, NPAD, D), jnp.float32),
    scratch_types=[
        pltpu.VMEM((2, NB, CH), jnp.int32),
        pltpu.VMEM((2, NB, CH), jnp.int32),
    ] + [pltpu.VMEM((CH, D), jnp.float32) for _ in range(NB)] + [
        pltpu.VMEM((16, D), jnp.float32),
        pltpu.VMEM_SHARED((NPAD, D), jnp.float32),
    ] + [pltpu.SemaphoreType.DMA for _ in range(NB + 2)],
    compiler_params=pltpu.CompilerParams(use_tc_tiling_on_sc=False),
)(_sc_agg_body_nodeg)


def _tc_body(sums_ref, degs_ref, x_ref, a_ref, b_ref, bias_ref, out_ref, *,
             relu):
    s = sums_ref[0] + sums_ref[1]
    deg = jnp.maximum(degs_ref[0, :, :1] + degs_ref[1, :, :1], 1.0)
    m = s / deg
    y = jnp.dot(m, a_ref[:], preferred_element_type=jnp.float32)
    y = y + jnp.dot(x_ref[:], b_ref[:], preferred_element_type=jnp.float32)
    y = y + bias_ref[:]
    if relu:
        y = jnp.maximum(y, 0.0)
    out_ref[:] = y


def _dense_layer(sums, degs, x, a_w, b_w, bias, relu):
    br = 200
    return pl.pallas_call(
        functools.partial(_tc_body, relu=relu),
        grid=(N // br,),
        in_specs=[
            pl.BlockSpec((NC, br, D), lambda i: (0, i, 0)),
            pl.BlockSpec((NC, br, DG), lambda i: (0, i, 0)),
            pl.BlockSpec((br, D), lambda i: (i, 0)),
            pl.BlockSpec((D, D), lambda i: (0, 0)),
            pl.BlockSpec((D, D), lambda i: (0, 0)),
            pl.BlockSpec((1, D), lambda i: (0, 0)),
        ],
        out_specs=pl.BlockSpec((br, D), lambda i: (i, 0)),
        out_shape=jax.ShapeDtypeStruct((N, D), jnp.float32),
    )(sums, degs, x, a_w, b_w, bias)


def kernel(x, edge_index, W0l, W0r, b0, bn_gamma, bn_beta, bn_mean, bn_var, W1l, W1r, b1):
    src = edge_index[0]
    dst = edge_index[1]

    g = bn_gamma / jnp.sqrt(bn_var + 1e-5)
    c = bn_beta - bn_mean * g
    a0 = W0l.T * g
    b0w = W0r.T * g
    bias0 = (b0 * g + c)[None, :]
    a1 = W1l.T
    b1w = W1r.T
    bias1 = b1[None, :]

    zeros = jnp.zeros((CH, D), jnp.float32)
    ones16 = jnp.ones((CH, DG), jnp.float32)
    zeros16 = jnp.zeros((CH, DG), jnp.float32)

    pad_e = EPAD - E
    src_c = jnp.concatenate(
        [src, jnp.zeros((pad_e,), jnp.int32)]).reshape(TOTCH, CH)
    dst_c = jnp.concatenate(
        [dst, jnp.full((pad_e,), NPAD - 1, jnp.int32)]).reshape(TOTCH, CH)

    sums0, degs = _sc_agg_deg(x, src_c, dst_c, zeros, ones16, zeros16)
    h = _dense_layer(sums0, degs, x, a0, b0w, bias0, relu=True)
    sums1 = _sc_agg_nodeg(h, src_c, dst_c, zeros)
    logit = _dense_layer(sums1, degs, h, a1, b1w, bias1, relu=False)

    return (logit, h)
